# baseline (speedup 1.0000x reference)
import functools

import jax
import jax.numpy as jnp
from jax import lax
from jax.experimental import pallas as pl
from jax.experimental.pallas import tpu as pltpu
from jax.experimental.pallas import tpu_sc as plsc

SEQ = 50
D = 128
BATCH = 4096
NC, NS = 2, 16
NW = NC * NS
CHUNK = 400
SUB = 80
NSUB = CHUNK // SUB
LP = SEQ * D // 16
ROWS_PER_P = CHUNK // SEQ
NSTAGE = 2

_mesh = plsc.VectorSubcoreMesh(
    core_axis_name="c", subcore_axis_name="s", num_cores=NC, num_subcores=NS
)


def _make_stage(nbatch):
    tot = nbatch * SEQ
    per_w = tot // NW
    nchunk = per_w // CHUNK

    @functools.partial(
        pl.kernel,
        out_type=jax.ShapeDtypeStruct((nbatch, SEQ, D), jnp.float32),
        mesh=_mesh,
        scratch_types=[
            pltpu.VMEM((nchunk, NSUB, SUB), jnp.int32),
            pltpu.VMEM((2, CHUNK, D), jnp.float32),
            pltpu.VMEM((SEQ * D,), jnp.float32),
            pltpu.SemaphoreType.DMA,
            pltpu.SemaphoreType.DMA,
            pltpu.SemaphoreType.DMA,
            pltpu.SemaphoreType.DMA,
        ],
    )
    def _sc_embed(idx_hbm, table_hbm, pos_hbm, out_hbm, idx_v, rows_v, pos_v,
                  gsem0, gsem1, osem0, osem1):
        wid = lax.axis_index("s") * NC + lax.axis_index("c")
        base = wid * per_w
        cbase = wid * nchunk
        gsems = (gsem0, gsem1)
        osems = (osem0, osem1)

        pltpu.sync_copy(pos_hbm, pos_v)
        pltpu.sync_copy(idx_hbm.at[pl.ds(cbase, nchunk)], idx_v)

        def fire_gather(c, slot):
            return [
                pltpu.async_copy(
                    table_hbm.at[idx_v.at[c, j]],
                    rows_v.at[slot, pl.ds(j * SUB, SUB)],
                    gsems[slot],
                )
                for j in range(NSUB)
            ]

        def add_pos(slot):
            rows = rows_v.at[slot]

            def body(p, carry):
                pv = pos_v[pl.ds(p * 16, 16)]
                r = p // (D // 16)
                dc = (p % (D // 16)) * 16
                for k in range(ROWS_PER_P):
                    plsc.addupdate(rows.at[k * SEQ + r, pl.ds(dc, 16)], pv)
                return carry

            lax.fori_loop(0, LP, body, 0, unroll=4)

        gh = {0: fire_gather(0, 0)}
        oh = {}
        for c in range(nchunk):
            slot = c % 2
            if c + 1 < nchunk:
                if c - 1 >= 0:
                    for h in oh[c - 1]:
                        h.wait()
                gh[c + 1] = fire_gather(c + 1, 1 - slot)
            for h in gh.pop(c):
                h.wait()
            add_pos(slot)
            b0 = (base + c * CHUNK) // SEQ
            oh[c] = [
                pltpu.async_copy(
                    rows_v.at[slot, pl.ds(k * SEQ, SEQ)],
                    out_hbm.at[b0 + k],
                    osems[slot],
                )
                for k in range(CHUNK // SEQ)
            ]
        for h in oh[nchunk - 2]:
            h.wait()
        for h in oh[nchunk - 1]:
            h.wait()

    return _sc_embed


_stage = _make_stage(BATCH // NSTAGE)


def kernel(inputs, word_table, pos_table):
    pos_flat = jnp.reshape(pos_table, (SEQ * D,))
    sb = BATCH // NSTAGE
    nchunk = sb * SEQ // NW // CHUNK
    parts = []
    for s in range(NSTAGE):
        idx = jnp.reshape(
            inputs[s * sb:(s + 1) * sb].astype(jnp.int32),
            (NW * nchunk, NSUB, SUB),
        )
        parts.append(_stage(idx, word_table, pos_flat))
    return jnp.concatenate(parts, axis=0)


# speedup vs baseline: 1.5834x; 1.5834x over previous
import functools

import jax
import jax.numpy as jnp
from jax import lax
from jax.experimental import pallas as pl
from jax.experimental.pallas import tpu as pltpu
from jax.experimental.pallas import tpu_sc as plsc

SEQ = 50
D = 128
BATCH = 4096
NC, NS = 2, 16
NW = NC * NS
CHUNK = 400
SUB = 80
NSUB = CHUNK // SUB
LP = SEQ * D // 16
ROWS_PER_P = CHUNK // SEQ
NSTAGE = 1

_mesh = plsc.VectorSubcoreMesh(
    core_axis_name="c", subcore_axis_name="s", num_cores=NC, num_subcores=NS
)


def _make_stage(nbatch):
    tot = nbatch * SEQ
    per_w = tot // NW
    nchunk = per_w // CHUNK

    @functools.partial(
        pl.kernel,
        out_type=jax.ShapeDtypeStruct((nbatch, SEQ, D), jnp.float32),
        mesh=_mesh,
        compiler_params=pltpu.CompilerParams(needs_layout_passes=True),
        scratch_types=[
            pltpu.VMEM((nchunk, NSUB, SUB), jnp.int32),
            pltpu.VMEM((2, CHUNK, D), jnp.float32),
            pltpu.VMEM((SEQ * D,), jnp.float32),
            pltpu.SemaphoreType.DMA,
            pltpu.SemaphoreType.DMA,
            pltpu.SemaphoreType.DMA,
            pltpu.SemaphoreType.DMA,
        ],
    )
    def _sc_embed(idx_hbm, table_hbm, pos_hbm, out_hbm, idx_v, rows_v, pos_v,
                  gsem0, gsem1, osem0, osem1):
        wid = lax.axis_index("s") * NC + lax.axis_index("c")
        base = wid * per_w
        cbase = wid * nchunk
        gsems = (gsem0, gsem1)
        osems = (osem0, osem1)

        pltpu.sync_copy(pos_hbm, pos_v)
        pltpu.sync_copy(idx_hbm.at[pl.ds(cbase, nchunk)], idx_v)

        def fire_gather(c, slot):
            return [
                pltpu.async_copy(
                    table_hbm.at[idx_v.at[c, j]],
                    rows_v.at[slot, pl.ds(j * SUB, SUB)],
                    gsems[slot],
                )
                for j in range(NSUB)
            ]

        def add_pos(slot):
            rows = rows_v.at[slot]

            def body(p, carry):
                pv = pos_v[pl.ds(p * 16, 16)]
                r = p // (D // 16)
                dc = (p % (D // 16)) * 16
                for k in range(ROWS_PER_P):
                    plsc.addupdate(rows.at[k * SEQ + r, pl.ds(dc, 16)], pv)
                return carry

            lax.fori_loop(0, LP, body, 0, unroll=4)

        gh = {0: fire_gather(0, 0)}
        oh = {}
        for c in range(nchunk):
            slot = c % 2
            if c + 1 < nchunk:
                if c - 1 >= 0:
                    for h in oh[c - 1]:
                        h.wait()
                gh[c + 1] = fire_gather(c + 1, 1 - slot)
            for h in gh.pop(c):
                h.wait()
            add_pos(slot)
            b0 = (base + c * CHUNK) // SEQ
            oh[c] = [
                pltpu.async_copy(
                    rows_v.at[slot, pl.ds(k * SEQ, SEQ)],
                    out_hbm.at[b0 + k],
                    osems[slot],
                )
                for k in range(CHUNK // SEQ)
            ]
        for h in oh[nchunk - 2]:
            h.wait()
        for h in oh[nchunk - 1]:
            h.wait()

    return _sc_embed


_stage = _make_stage(BATCH // NSTAGE)


def kernel(inputs, word_table, pos_table):
    pos_flat = jnp.reshape(pos_table, (SEQ * D,))
    sb = BATCH // NSTAGE
    nchunk = sb * SEQ // NW // CHUNK
    parts = []
    for s in range(NSTAGE):
        idx = jnp.reshape(
            inputs[s * sb:(s + 1) * sb].astype(jnp.int32),
            (NW * nchunk, NSUB, SUB),
        )
        parts.append(_stage(idx, word_table, pos_flat))
    return jnp.concatenate(parts, axis=0)


# speedup vs baseline: 1.6557x; 1.0457x over previous
import functools

import jax
import jax.numpy as jnp
from jax import lax
from jax.experimental import pallas as pl
from jax.experimental.pallas import tpu as pltpu
from jax.experimental.pallas import tpu_sc as plsc

SEQ = 50
D = 128
BATCH = 4096
NC, NS = 2, 16
NW = NC * NS
TOT = BATCH * SEQ
PER_W = TOT // NW
CHUNK = 200
NCHUNK = PER_W // CHUNK
SUBS = (104, 96)
NRING = 4
ROWS_PER_P = CHUNK // SEQ
BPC = CHUNK // SEQ

_mesh = plsc.VectorSubcoreMesh(
    core_axis_name="c", subcore_axis_name="s", num_cores=NC, num_subcores=NS
)


@functools.partial(
    pl.kernel,
    out_type=jax.ShapeDtypeStruct((BATCH, SEQ, D), jnp.float32),
    mesh=_mesh,
    scratch_types=[
        pltpu.VMEM((NCHUNK * CHUNK,), jnp.int32),
        pltpu.VMEM((NRING, CHUNK, D), jnp.float32),
        pltpu.VMEM((SEQ * D,), jnp.float32),
        [pltpu.SemaphoreType.DMA] * NRING,
        [pltpu.SemaphoreType.DMA] * NRING,
    ],
)
def _sc_embed(idx_hbm, table_hbm, pos_hbm, out_hbm, idx_v, rows_v, pos_v,
              gsems, osems):
    wid = lax.axis_index("s") * NC + lax.axis_index("c")
    base = wid * PER_W

    pltpu.sync_copy(pos_hbm, pos_v)
    pltpu.sync_copy(idx_hbm.at[pl.ds(base, NCHUNK * CHUNK)], idx_v)

    def fire_gather(c):
        slot = c % NRING
        hs = []
        off = 0
        for sub in SUBS:
            hs.append(pltpu.async_copy(
                table_hbm.at[idx_v.at[pl.ds(c * CHUNK + off, sub)]],
                rows_v.at[slot, pl.ds(off, sub)],
                gsems[slot],
            ))
            off += sub
        return hs

    def add_pos(slot):
        rows = rows_v.at[slot]
        nd = D // 16

        def body(r, carry):
            pv = [pos_v[pl.ds(r * D + d * 16, 16)] for d in range(nd)]
            for k in range(ROWS_PER_P):
                for d in range(nd):
                    plsc.addupdate(
                        rows.at[k * SEQ + r, pl.ds(d * 16, 16)], pv[d]
                    )
            return carry

        lax.fori_loop(0, SEQ, body, 0, unroll=1)

    def fire_out(c):
        slot = c % NRING
        b0 = (base + c * CHUNK) // SEQ
        return [
            pltpu.async_copy(
                rows_v.at[slot, pl.ds(k * SEQ, SEQ)],
                out_hbm.at[b0 + k],
                osems[slot],
            )
            for k in range(BPC)
        ]

    gh = {0: fire_gather(0), 1: fire_gather(1)}
    oh = {}
    for c in range(NCHUNK):
        if c + 2 < NCHUNK:
            if c - 2 >= 0:
                for h in oh[c - 2]:
                    h.wait()
            gh[c + 2] = fire_gather(c + 2)
        for h in gh.pop(c):
            h.wait()
        add_pos(c % NRING)
        oh[c] = fire_out(c)
    for c in range(NCHUNK - 4, NCHUNK):
        for h in oh[c]:
            h.wait()


def kernel(inputs, word_table, pos_table):
    idx = jnp.reshape(inputs.astype(jnp.int32), (TOT,))
    pos_flat = jnp.reshape(pos_table, (SEQ * D,))
    return _sc_embed(idx, word_table, pos_flat)


# speedup vs baseline: 1.7119x; 1.0340x over previous
import functools

import jax
import jax.numpy as jnp
from jax import lax
from jax.experimental import pallas as pl
from jax.experimental.pallas import tpu as pltpu
from jax.experimental.pallas import tpu_sc as plsc

SEQ = 50
D = 128
BATCH = 4096
NC, NS = 2, 16
NW = NC * NS
TOT = BATCH * SEQ
PER_W = TOT // NW
CHUNK = 200
NCHUNK = PER_W // CHUNK
SUBS = (104, 96)
NRING = 4
ROWS_PER_P = CHUNK // SEQ
BPC = CHUNK // SEQ

_mesh = plsc.VectorSubcoreMesh(
    core_axis_name="c", subcore_axis_name="s", num_cores=NC, num_subcores=NS
)


@functools.partial(
    pl.kernel,
    out_type=jax.ShapeDtypeStruct((BATCH, SEQ, D), jnp.float32),
    mesh=_mesh,
    scratch_types=[
        pltpu.VMEM((NCHUNK * CHUNK,), jnp.int32),
        pltpu.VMEM((NRING, CHUNK, D), jnp.float32),
        pltpu.VMEM((SEQ * D,), jnp.float32),
        [pltpu.SemaphoreType.DMA] * NRING,
        [pltpu.SemaphoreType.DMA] * NRING,
    ],
)
def _sc_embed(idx_hbm, table_hbm, pos_hbm, out_hbm, idx_v, rows_v, pos_v,
              gsems, osems):
    wid = lax.axis_index("s") * NC + lax.axis_index("c")
    base = wid * PER_W

    pltpu.sync_copy(pos_hbm, pos_v)
    pltpu.sync_copy(idx_hbm.at[pl.ds(base, NCHUNK * CHUNK)], idx_v)

    def fire_gather(c, slot):
        off = 0
        for sub in SUBS:
            pltpu.async_copy(
                table_hbm.at[idx_v.at[pl.ds(pl.multiple_of(c * CHUNK + off, 8), sub)]],
                rows_v.at[slot, pl.ds(off, sub)],
                gsems[slot],
            )
            off += sub

    def add_pos(slot):
        rows = rows_v.at[slot]
        nd = D // 16

        def body(r, carry):
            pv = [pos_v[pl.ds(r * D + d * 16, 16)] for d in range(nd)]
            for k in range(ROWS_PER_P):
                for d in range(nd):
                    plsc.addupdate(
                        rows.at[k * SEQ + r, pl.ds(d * 16, 16)], pv[d]
                    )
            return carry

        lax.fori_loop(0, SEQ, body, 0, unroll=1)

    def fire_out(c, slot):
        b0 = (base + c * CHUNK) // SEQ
        for k in range(BPC):
            pltpu.async_copy(
                rows_v.at[slot, pl.ds(k * SEQ, SEQ)],
                out_hbm.at[b0 + k],
                osems[slot],
            )

    def wait_gather(c, slot):
        off = 0
        for sub in SUBS:
            pltpu.make_async_copy(
                table_hbm.at[idx_v.at[pl.ds(pl.multiple_of(c * CHUNK + off, 8), sub)]],
                rows_v.at[slot, pl.ds(off, sub)],
                gsems[slot],
            ).wait()
            off += sub

    def wait_out(c, slot):
        b0 = (base + c * CHUNK) // SEQ
        for k in range(BPC):
            pltpu.make_async_copy(
                rows_v.at[slot, pl.ds(k * SEQ, SEQ)],
                out_hbm.at[b0 + k],
                osems[slot],
            ).wait()

    fire_gather(0, 0)
    fire_gather(1, 1)

    def group(p, carry):
        for i in range(NRING):
            c = p * NRING + i

            @pl.when(c + 2 < NCHUNK)
            def _():
                @pl.when(c >= 2)
                def _():
                    wait_out(c - 2, (i - 2) % NRING)
                fire_gather(c + 2, (i + 2) % NRING)

            wait_gather(c, i)
            add_pos(i)
            fire_out(c, i)
        return carry

    lax.fori_loop(0, NCHUNK // NRING, group, 0, unroll=1)
    for c in range(NCHUNK - 4, NCHUNK):
        wait_out(c, c % NRING)


def kernel(inputs, word_table, pos_table):
    idx = jnp.reshape(inputs.astype(jnp.int32), (TOT,))
    pos_flat = jnp.reshape(pos_table, (SEQ * D,))
    return _sc_embed(idx, word_table, pos_flat)
